# Initial kernel scaffold; baseline (speedup 1.0000x reference)
import jax
import jax.numpy as jnp
from jax.experimental import pallas as pl


def kernel(seq, vocabulary):
    raise NotImplementedError("write your pallas kernel here")


# speedup vs baseline: 1.0463x; 1.0463x over previous
import functools

import jax
import jax.numpy as jnp
from jax import lax
from jax.experimental import pallas as pl
from jax.experimental.pallas import tpu as pltpu
from jax.experimental.pallas import tpu_sc as plsc

_TM = 512


def _argmin_body(a16_ref, b_ref, a2_ref, b2_ref, idx_ref):
    a16 = a16_ref[...]
    b = b_ref[...]
    a2 = a2_ref[...]
    b2 = b2_ref[...]
    ab = lax.dot_general(a16, b, (((1,), (1,)), ((), ())),
                         preferred_element_type=jnp.float32)
    d2 = jnp.maximum((a2 - 2.0 * ab) + b2, 0.0)
    s = jnp.sqrt(d2)
    m = jnp.min(s, axis=1, keepdims=True)
    n = s.shape[1]
    iota = lax.broadcasted_iota(jnp.int32, s.shape, 1)
    idx = jnp.min(jnp.where(s == m, iota, jnp.int32(n)), axis=1,
                  keepdims=True)
    idx_ref[...] = idx


def _nearest_codes(a16, vocabulary, a2, b2):
    m, k = a16.shape
    n, _ = vocabulary.shape
    return pl.pallas_call(
        _argmin_body,
        grid=(m // _TM,),
        in_specs=[
            pl.BlockSpec((_TM, k), lambda i: (i, 0)),
            pl.BlockSpec((n, k), lambda i: (0, 0)),
            pl.BlockSpec((_TM, 1), lambda i: (i, 0)),
            pl.BlockSpec((1, n), lambda i: (0, 0)),
        ],
        out_specs=pl.BlockSpec((_TM, 1), lambda i: (i, 0)),
        out_shape=jax.ShapeDtypeStruct((m, 1), jnp.int32),
    )(a16, vocabulary, a2, b2)


_CH = 128


def _make_sc_gather(v, d, b):
    info = plsc.get_sparse_core_info()
    nw = info.num_cores * info.num_subcores
    bpw = b // nw
    nch = bpw // _CH
    assert b % nw == 0 and bpw % _CH == 0 and bpw % 8 == 0
    mesh = plsc.VectorSubcoreMesh(core_axis_name="c", subcore_axis_name="s")

    @functools.partial(
        pl.kernel,
        mesh=mesh,
        compiler_params=pltpu.CompilerParams(use_tc_tiling_on_sc=False),
        out_type=jax.ShapeDtypeStruct((b, d), jnp.float32),
        scratch_types=[
            pltpu.VMEM((nch, _CH), jnp.int32),
            pltpu.VMEM((bpw, d), jnp.float32),
            pltpu.SemaphoreType.DMA,
        ],
    )
    def gather(table_hbm, idx_hbm, out_hbm, idx_v, rows_v, sem):
        wid = lax.axis_index("s") * info.num_cores + lax.axis_index("c")
        for c in range(nch):
            pltpu.sync_copy(idx_hbm.at[pl.ds(wid * bpw + c * _CH, _CH)],
                            idx_v.at[c])
        copies = [
            pltpu.async_copy(
                table_hbm.at[idx_v.at[c]],
                rows_v.at[pl.ds(c * _CH, _CH)],
                sem,
            )
            for c in range(nch)
        ]
        for cp in copies:
            cp.wait()
        pltpu.sync_copy(rows_v, out_hbm.at[pl.ds(wid * bpw, bpw)])

    return gather, nw, nch


def kernel(seq, vocabulary):
    n_seq, bs, dim = seq.shape
    flat = seq.reshape(n_seq * bs, dim)
    m = flat.shape[0]
    a16 = flat.astype(jnp.bfloat16)
    a2 = jnp.sum(flat * flat, axis=1, keepdims=True)
    b2 = jnp.sum(vocabulary * vocabulary, axis=1)[None, :]
    idx = _nearest_codes(a16, vocabulary, a2, b2).reshape(m)
    sc_gather, nw, nch = _make_sc_gather(vocabulary.shape[0], dim, m)
    enc = sc_gather(vocabulary, idx)
    return enc.reshape(n_seq, bs, dim), idx.reshape(n_seq, bs)


# speedup vs baseline: 1.1259x; 1.0761x over previous
import functools

import jax
import jax.numpy as jnp
from jax import lax
from jax.experimental import pallas as pl
from jax.experimental.pallas import tpu as pltpu
from jax.experimental.pallas import tpu_sc as plsc

_TM = 1024


def _argmin_body(a16_ref, b_ref, a2_ref, b2_ref, idx_ref):
    a16 = a16_ref[...]
    b = b_ref[...]
    a2 = a2_ref[...]
    b2 = b2_ref[...]
    ab = lax.dot_general(a16, b, (((1,), (1,)), ((), ())),
                         preferred_element_type=jnp.float32)
    d2 = jnp.maximum((a2 - 2.0 * ab) + b2, 0.0)
    s = jnp.sqrt(d2)
    m = jnp.min(s, axis=1, keepdims=True)
    n = s.shape[1]
    iota = lax.broadcasted_iota(jnp.int32, s.shape, 1)
    idx = jnp.min(jnp.where(s == m, iota, jnp.int32(n)), axis=1,
                  keepdims=True)
    idx_ref[...] = idx


def _nearest_codes(a16, vocabulary, a2, b2):
    m, k = a16.shape
    n, _ = vocabulary.shape
    return pl.pallas_call(
        _argmin_body,
        grid=(m // _TM,),
        in_specs=[
            pl.BlockSpec((_TM, k), lambda i: (i, 0)),
            pl.BlockSpec((n, k), lambda i: (0, 0)),
            pl.BlockSpec((_TM, 1), lambda i: (i, 0)),
            pl.BlockSpec((1, n), lambda i: (0, 0)),
        ],
        out_specs=pl.BlockSpec((_TM, 1), lambda i: (i, 0)),
        out_shape=jax.ShapeDtypeStruct((m, 1), jnp.int32),
    )(a16, vocabulary, a2, b2)


_CH = 128


def _make_sc_gather(v, d, b):
    info = plsc.get_sparse_core_info()
    nw = info.num_cores * info.num_subcores
    bpw = b // nw
    nch = bpw // _CH
    assert b % nw == 0 and bpw % _CH == 0 and bpw % 8 == 0
    mesh = plsc.VectorSubcoreMesh(core_axis_name="c", subcore_axis_name="s")

    @functools.partial(
        pl.kernel,
        mesh=mesh,
        compiler_params=pltpu.CompilerParams(use_tc_tiling_on_sc=False),
        out_type=jax.ShapeDtypeStruct((b, d), jnp.float32),
        scratch_types=[
            pltpu.VMEM((nch, _CH), jnp.int32),
            pltpu.VMEM((bpw, d), jnp.float32),
            pltpu.SemaphoreType.DMA,
        ],
    )
    def gather(table_hbm, idx_hbm, out_hbm, idx_v, rows_v, sem):
        wid = lax.axis_index("s") * info.num_cores + lax.axis_index("c")
        for c in range(nch):
            pltpu.sync_copy(idx_hbm.at[pl.ds(wid * bpw + c * _CH, _CH)],
                            idx_v.at[c])
        copies = [
            pltpu.async_copy(
                table_hbm.at[idx_v.at[c]],
                rows_v.at[pl.ds(c * _CH, _CH)],
                sem,
            )
            for c in range(nch)
        ]
        for cp in copies:
            cp.wait()
        pltpu.sync_copy(rows_v, out_hbm.at[pl.ds(wid * bpw, bpw)])

    return gather, nw, nch


def kernel(seq, vocabulary):
    n_seq, bs, dim = seq.shape
    flat = seq.reshape(n_seq * bs, dim)
    m = flat.shape[0]
    a16 = flat.astype(jnp.bfloat16)
    a2 = jnp.sum(flat * flat, axis=1, keepdims=True)
    b2 = jnp.sum(vocabulary * vocabulary, axis=1)[None, :]
    idx = _nearest_codes(a16, vocabulary, a2, b2).reshape(m)
    sc_gather, nw, nch = _make_sc_gather(vocabulary.shape[0], dim, m)
    enc = sc_gather(vocabulary, idx)
    return enc.reshape(n_seq, bs, dim), idx.reshape(n_seq, bs)
